# baseline (speedup 1.0000x reference)
import jax
import jax.numpy as jnp
from jax.experimental import pallas as pl
from jax.experimental.pallas import tpu as pltpu

_B, _N, _G, _C = 16, 8192, 512, 64
_CHUNK = 4096


def _fps_body(x_ref, y_ref, z_ref, cx_ref, cy_ref, cz_ref, dists_ref):
    x = x_ref[...]
    y = y_ref[...]
    z = z_ref[...]
    dists_ref[...] = jnp.full((_B, _N), 1e10, dtype=jnp.float32)
    gi = jax.lax.broadcasted_iota(jnp.int32, (_B, _G), 1)

    lx0 = x[:, 0:1]
    ly0 = y[:, 0:1]
    lz0 = z[:, 0:1]
    cx0 = jnp.where(gi == 0, lx0, 0.0)
    cy0 = jnp.where(gi == 0, ly0, 0.0)
    cz0 = jnp.where(gi == 0, lz0, 0.0)

    def body(i, carry):
        lx, ly, lz, cx, cy, cz = carry
        dx = x - lx
        dy = y - ly
        dz = z - lz
        sx = dx * dx
        sy = dy * dy
        sz = dz * dz
        d = (sx + sy) + sz
        dn = jnp.minimum(dists_ref[...], d)
        dists_ref[...] = dn
        m = jnp.max(dn, axis=1, keepdims=True)
        iota = jax.lax.broadcasted_iota(jnp.int32, (_B, _N), 1)
        hit = dn == m
        idx = jnp.min(jnp.where(hit, iota, _N), axis=1, keepdims=True)
        sel = iota == idx
        ninf = jnp.float32(-jnp.inf)
        nlx = jnp.max(jnp.where(sel, x, ninf), axis=1, keepdims=True)
        nly = jnp.max(jnp.where(sel, y, ninf), axis=1, keepdims=True)
        nlz = jnp.max(jnp.where(sel, z, ninf), axis=1, keepdims=True)
        upd = gi == i
        cx = jnp.where(upd, nlx, cx)
        cy = jnp.where(upd, nly, cy)
        cz = jnp.where(upd, nlz, cz)
        return (nlx, nly, nlz, cx, cy, cz)

    carry = (lx0, ly0, lz0, cx0, cy0, cz0)
    _, _, _, cx, cy, cz = jax.lax.fori_loop(1, _G, body, carry)
    cx_ref[...] = cx
    cy_ref[...] = cy
    cz_ref[...] = cz


def _fps_call(x, y, z):
    f32 = jnp.float32
    return pl.pallas_call(
        _fps_body,
        out_shape=(
            jax.ShapeDtypeStruct((_B, _G), f32),
            jax.ShapeDtypeStruct((_B, _G), f32),
            jax.ShapeDtypeStruct((_B, _G), f32),
        ),
        scratch_shapes=[pltpu.VMEM((_B, _N), f32)],
    )(x, y, z)


def _group_body(xyz_ref, feat_ref, ct_ref, out_ref, idx_ref):
    q = xyz_ref[0]
    ct = ct_ref[0]
    cxr = ct[0:1, :]
    cyr = ct[1:2, :]
    czr = ct[2:3, :]
    k2 = (cxr * cxr + cyr * cyr) + czr * czr
    qx = q[:, 0:1]
    qy = q[:, 1:2]
    qz = q[:, 2:3]
    q2 = (qx * qx + qy * qy) + qz * qz
    qk2 = jax.lax.dot_general(
        q, ct * -2.0, (((1,), (0,)), ((), ())),
        preferred_element_type=jnp.float32)
    d2 = (q2 + k2) + qk2
    m = jnp.min(d2, axis=1, keepdims=True)
    gio = jax.lax.broadcasted_iota(jnp.int32, (_CHUNK, _G), 1)
    hit = d2 == m
    idx = jnp.min(jnp.where(hit, gio, _G), axis=1, keepdims=True)
    sel = jax.lax.dot_general(
        hit.astype(jnp.float32), ct, (((1,), (1,)), ((), ())),
        preferred_element_type=jnp.float32,
        precision=jax.lax.Precision.HIGHEST)
    nbr = q - sel
    nx = nbr[:, 0:1]
    ny = nbr[:, 1:2]
    nz = nbr[:, 2:3]
    dist = jnp.sqrt((nx * nx + ny * ny) + nz * nz)
    dn = jnp.maximum(dist, 1e-8)
    out_ref[0, :, 0:3] = nbr / dn
    out_ref[0, :, 3:4] = dist
    out_ref[0, :, 4:] = feat_ref[0]
    idx_ref[0] = idx


def _group_call(xyz, features, ct):
    grid = (_B, _N // _CHUNK)
    return pl.pallas_call(
        _group_body,
        grid=grid,
        in_specs=[
            pl.BlockSpec((1, _CHUNK, 3), lambda b, n: (b, n, 0)),
            pl.BlockSpec((1, _CHUNK, _C), lambda b, n: (b, n, 0)),
            pl.BlockSpec((1, 3, _G), lambda b, n: (b, 0, 0)),
        ],
        out_specs=[
            pl.BlockSpec((1, _CHUNK, 4 + _C), lambda b, n: (b, n, 0)),
            pl.BlockSpec((1, _CHUNK, 1), lambda b, n: (b, n, 0)),
        ],
        out_shape=(
            jax.ShapeDtypeStruct((_B, _N, 4 + _C), jnp.float32),
            jax.ShapeDtypeStruct((_B, _N, 1), jnp.int32),
        ),
    )(xyz, features, ct)


def kernel(xyz, features):
    x = xyz[:, :, 0]
    y = xyz[:, :, 1]
    z = xyz[:, :, 2]
    cx, cy, cz = _fps_call(x, y, z)
    centers = jnp.stack([cx, cy, cz], axis=-1)
    ct = jnp.stack([cx, cy, cz], axis=1)
    group_feats, idx = _group_call(xyz, features, ct)
    return (group_feats, centers, idx[:, :, 0])


# speedup vs baseline: 1.2633x; 1.2633x over previous
import jax
import jax.numpy as jnp
from jax.experimental import pallas as pl
from jax.experimental.pallas import tpu as pltpu

_B, _N, _G, _C = 16, 8192, 512, 64
_CHUNK = 2048
_GB = 1


_CW = 128
_NCH = _N // _CW


def _fps_body(x_ref, y_ref, z_ref, cx_ref, cy_ref, cz_ref, dists_ref):
    dists_ref[...] = jnp.full((_B, _N), 1e10, dtype=jnp.float32)
    gi = jax.lax.broadcasted_iota(jnp.int32, (_B, _G), 1)
    iota0 = jax.lax.broadcasted_iota(jnp.int32, (_B, _CW), 1)

    lx0 = x_ref[:, 0:1]
    ly0 = y_ref[:, 0:1]
    lz0 = z_ref[:, 0:1]
    cx0 = jnp.where(gi == 0, lx0, 0.0)
    cy0 = jnp.where(gi == 0, ly0, 0.0)
    cz0 = jnp.where(gi == 0, lz0, 0.0)
    ninf = jnp.float32(-jnp.inf)

    nacc = 4

    def body(i, carry):
        lx, ly, lz, cx, cy, cz = carry
        pms = [jnp.full((_B, _CW), ninf, dtype=jnp.float32)] * nacc
        for c in range(_NCH):
            sl = pl.ds(c * _CW, _CW)
            dxc = x_ref[:, sl] - lx
            dyc = y_ref[:, sl] - ly
            dzc = z_ref[:, sl] - lz
            sxc = dxc * dxc
            syc = dyc * dyc
            szc = dzc * dzc
            dc = (sxc + syc) + szc
            dnc = jnp.minimum(dists_ref[:, sl], dc)
            dists_ref[:, sl] = dnc
            pms[c % nacc] = jnp.maximum(pms[c % nacc], dnc)
        pm = jnp.maximum(jnp.maximum(pms[0], pms[1]),
                         jnp.maximum(pms[2], pms[3]))
        m = jnp.max(pm, axis=1, keepdims=True)
        pidxs = [jnp.full((_B, _CW), _N, dtype=jnp.int32)] * nacc
        pxs = [jnp.zeros((_B, _CW), dtype=jnp.float32)] * nacc
        pys = [jnp.zeros((_B, _CW), dtype=jnp.float32)] * nacc
        pzs = [jnp.zeros((_B, _CW), dtype=jnp.float32)] * nacc
        for c in range(_NCH):
            sl = pl.ds(c * _CW, _CW)
            k = c % nacc
            hitc = dists_ref[:, sl] == m
            wi = jnp.where(hitc, iota0 + (c * _CW), _N)
            upd = wi < pidxs[k]
            pidxs[k] = jnp.where(upd, wi, pidxs[k])
            pxs[k] = jnp.where(upd, x_ref[:, sl], pxs[k])
            pys[k] = jnp.where(upd, y_ref[:, sl], pys[k])
            pzs[k] = jnp.where(upd, z_ref[:, sl], pzs[k])
        while len(pidxs) > 1:
            u = pidxs[1] < pidxs[0]
            pidxs[0] = jnp.where(u, pidxs[1], pidxs[0])
            pxs[0] = jnp.where(u, pxs[1], pxs[0])
            pys[0] = jnp.where(u, pys[1], pys[0])
            pzs[0] = jnp.where(u, pzs[1], pzs[0])
            pidxs = [pidxs[0]] + pidxs[2:]
            pxs = [pxs[0]] + pxs[2:]
            pys = [pys[0]] + pys[2:]
            pzs = [pzs[0]] + pzs[2:]
        pidx, px, py, pz = pidxs[0], pxs[0], pys[0], pzs[0]
        idx = jnp.min(pidx, axis=1, keepdims=True)
        sel = pidx == idx
        nlx = jnp.max(jnp.where(sel, px, ninf), axis=1, keepdims=True)
        nly = jnp.max(jnp.where(sel, py, ninf), axis=1, keepdims=True)
        nlz = jnp.max(jnp.where(sel, pz, ninf), axis=1, keepdims=True)
        upd2 = gi == i
        cx = jnp.where(upd2, nlx, cx)
        cy = jnp.where(upd2, nly, cy)
        cz = jnp.where(upd2, nlz, cz)
        return (nlx, nly, nlz, cx, cy, cz)

    carry = (lx0, ly0, lz0, cx0, cy0, cz0)
    _, _, _, cx, cy, cz = jax.lax.fori_loop(1, _G, body, carry)
    cx_ref[...] = cx
    cy_ref[...] = cy
    cz_ref[...] = cz


def _fps_call(x, y, z):
    f32 = jnp.float32
    return pl.pallas_call(
        _fps_body,
        out_shape=(
            jax.ShapeDtypeStruct((_B, _G), f32),
            jax.ShapeDtypeStruct((_B, _G), f32),
            jax.ShapeDtypeStruct((_B, _G), f32),
        ),
        scratch_shapes=[pltpu.VMEM((_B, _N), f32)],
    )(x, y, z)


def _group_body(xyz_ref, feat_ref, ct_ref, out_ref, idx_ref):
    q = xyz_ref[...]
    ct = ct_ref[...]
    cxr = ct[:, 0:1, :]
    cyr = ct[:, 1:2, :]
    czr = ct[:, 2:3, :]
    k2 = (cxr * cxr + cyr * cyr) + czr * czr
    qx = q[:, :, 0:1]
    qy = q[:, :, 1:2]
    qz = q[:, :, 2:3]
    q2 = (qx * qx + qy * qy) + qz * qz
    qk2 = jax.lax.dot_general(
        q, ct * -2.0, (((2,), (1,)), ((0,), (0,))),
        preferred_element_type=jnp.float32)
    d2 = (q2 + k2) + qk2
    m = jnp.min(d2, axis=2, keepdims=True)
    gio = jax.lax.broadcasted_iota(jnp.int32, (_GB, _CHUNK, _G), 2)
    hit = d2 == m
    idx = jnp.min(jnp.where(hit, gio, _G), axis=2, keepdims=True)
    onehot = (gio == idx).astype(jnp.float32)
    sel = jax.lax.dot_general(
        onehot, ct, (((2,), (2,)), ((0,), (0,))),
        preferred_element_type=jnp.float32,
        precision=jax.lax.Precision.HIGHEST)
    nbr = q - sel
    nx = nbr[:, :, 0:1]
    ny = nbr[:, :, 1:2]
    nz = nbr[:, :, 2:3]
    dist = jnp.sqrt((nx * nx + ny * ny) + nz * nz)
    dn = jnp.maximum(dist, 1e-8)
    out_ref[:, :, 0:3] = nbr / dn
    out_ref[:, :, 3:4] = dist
    out_ref[:, :, 4:] = feat_ref[...]
    idx_ref[...] = idx


def _group_call(xyz, features, ct):
    grid = (_B // _GB, _N // _CHUNK)
    return pl.pallas_call(
        _group_body,
        grid=grid,
        in_specs=[
            pl.BlockSpec((_GB, _CHUNK, 3), lambda b, n: (b, n, 0)),
            pl.BlockSpec((_GB, _CHUNK, _C), lambda b, n: (b, n, 0)),
            pl.BlockSpec((_GB, 3, _G), lambda b, n: (b, 0, 0)),
        ],
        out_specs=[
            pl.BlockSpec((_GB, _CHUNK, 4 + _C), lambda b, n: (b, n, 0)),
            pl.BlockSpec((_GB, _CHUNK, 1), lambda b, n: (b, n, 0)),
        ],
        out_shape=(
            jax.ShapeDtypeStruct((_B, _N, 4 + _C), jnp.float32),
            jax.ShapeDtypeStruct((_B, _N, 1), jnp.int32),
        ),
    )(xyz, features, ct)


def kernel(xyz, features):
    x = xyz[:, :, 0]
    y = xyz[:, :, 1]
    z = xyz[:, :, 2]
    cx, cy, cz = _fps_call(x, y, z)
    centers = jnp.stack([cx, cy, cz], axis=-1)
    ct = jnp.stack([cx, cy, cz], axis=1)
    group_feats, idx = _group_call(xyz, features, ct)
    return (group_feats, centers, idx[:, :, 0])
